# baseline (speedup 1.0000x reference)
import functools

import jax
import jax.numpy as jnp
from jax import lax
from jax.experimental import pallas as pl
from jax.experimental.pallas import tpu as pltpu
from jax.experimental.pallas import tpu_sc as plsc


@functools.cache
def _make_convert(V: int, D: int):
    info = plsc.get_sparse_core_info()
    NC, NS = info.num_cores, info.num_subcores
    NW = NC * NS
    n_full = V // 128
    n_tail = V - n_full * 128
    base_per_w = n_full // NW
    extra = n_full - base_per_w * NW
    mesh = plsc.VectorSubcoreMesh(core_axis_name="c", subcore_axis_name="s")

    @functools.partial(
        pl.kernel,
        mesh=mesh,
        out_type=jax.ShapeDtypeStruct((V // 2, 2 * D), jnp.float32),
        scratch_types=[
            pltpu.VMEM((D, 128), jnp.float32),
            pltpu.VMEM((D, 128), jnp.float32),
            pltpu.SemaphoreType.DMA,
        ],
        compiler_params=pltpu.CompilerParams(
            use_tc_tiling_on_sc=True, needs_layout_passes=False),
    )
    def k(wt_hbm, tail_hbm, t2_hbm, a_v, b_v, sem):
        wid = lax.axis_index("s") * NC + lax.axis_index("c")
        t0 = wid * base_per_w + jnp.minimum(wid, extra)
        nblk = base_per_w + jnp.where(wid < extra, 1, 0)

        def transpose_block(rows):
            def row_body(r, carry):
                for q in range(128 // 16):
                    col = 2 * r + (1 if (q * 16) >= D else 0)
                    lane0 = (q * 16) % D
                    vals = plsc.load_gather(
                        a_v,
                        [lane0 + lax.iota(jnp.int32, 16),
                         jnp.full((16,), col, jnp.int32)])
                    b_v[r, pl.ds(q * 16, 16)] = vals
                return carry
            lax.fori_loop(0, rows, row_body, 0, unroll=2)

        def blk_body(i, carry):
            t = t0 + i
            pltpu.async_copy(
                wt_hbm.at[:, pl.ds(t * 128, 128)], a_v, sem).wait()
            transpose_block(64)
            pltpu.async_copy(
                b_v, t2_hbm.at[pl.ds(t * 64, 64)], sem).wait()
            return carry

        lax.fori_loop(0, nblk, blk_body, 0)

        if n_tail:
            @pl.when(wid == NW - 1)
            def _tail():
                pltpu.async_copy(
                    tail_hbm.at[:], a_v.at[pl.ds(0, n_tail)], sem).wait()
                def row_body(r, carry):
                    for q in range(128 // 16):
                        row = 2 * r + (1 if (q * 16) >= D else 0)
                        lane0 = (q * 16) % D
                        vals = plsc.load_gather(
                            a_v,
                            [jnp.full((16,), row, jnp.int32),
                             lane0 + lax.iota(jnp.int32, 16)])
                        b_v[r, pl.ds(q * 16, 16)] = vals
                    return carry
                lax.fori_loop(0, n_tail // 2, row_body, 0)
                pltpu.async_copy(
                    b_v.at[pl.ds(0, n_tail // 2)],
                    t2_hbm.at[pl.ds(n_full * 64, n_tail // 2)], sem).wait()

    return k


@functools.cache
def _make_gather(B: int, C: int, NBUF: int):
    info = plsc.get_sparse_core_info()
    NC, NS = info.num_cores, info.num_subcores
    NW = NC * NS
    assert B % (NW * C) == 0
    b_per_w = B // NW
    n_chunks = b_per_w // C
    assert n_chunks % NBUF == 0
    n_groups = n_chunks // NBUF
    mesh = plsc.VectorSubcoreMesh(core_axis_name="c", subcore_axis_name="s")

    @functools.partial(
        pl.kernel,
        mesh=mesh,
        out_type=jax.ShapeDtypeStruct((B, 128), jnp.float32),
        scratch_types=(
            [pltpu.VMEM((C,), jnp.int32) for _ in range(NBUF)]
            + [pltpu.VMEM((C, 64), jnp.float32) for _ in range(NBUF)]
            + [pltpu.SemaphoreType.DMA] * (3 * NBUF)
        ),
        compiler_params=pltpu.CompilerParams(use_tc_tiling_on_sc=False),
    )
    def k(table_hbm, idx_hbm, out_hbm, *scratch):
        idx_v = scratch[:NBUF]
        rows_v = scratch[NBUF:2 * NBUF]
        sems = scratch[2 * NBUF:]
        sem_i, sem_g, sem_o = (
            sems[:NBUF], sems[NBUF:2 * NBUF], sems[2 * NBUF:])
        wid = lax.axis_index("s") * NC + lax.axis_index("c")
        base = wid * b_per_w

        def start_idx(j, b):
            pltpu.async_copy(
                idx_hbm.at[pl.ds(base + j * C, C)], idx_v[b], sem_i[b])

        def start_gather(b):
            pltpu.make_async_copy(
                idx_hbm.at[pl.ds(0, C)], idx_v[b], sem_i[b]).wait()
            pltpu.async_copy(
                table_hbm.at[idx_v[b]], rows_v[b], sem_g[b])

        def start_out(j, b):
            pltpu.make_async_copy(
                table_hbm.at[idx_v[b]], rows_v[b], sem_g[b]).wait()
            pltpu.async_copy(
                rows_v[b],
                out_hbm.at[pl.ds(base + j * C, C), pl.ds(0, 64)], sem_o[b])

        def wait_out(b):
            pltpu.make_async_copy(
                rows_v[b],
                out_hbm.at[pl.ds(0, C), pl.ds(0, 64)], sem_o[b]).wait()

        for b in range(NBUF):
            start_idx(b, b)
        for b in range(NBUF):
            start_gather(b)
        for b in range(NBUF):
            start_out(b, b)

        def body(g, carry):
            j0 = g * NBUF
            for b in range(NBUF):
                start_idx(j0 + b, b)
            for b in range(NBUF):
                wait_out(b)
                start_gather(b)
            for b in range(NBUF):
                start_out(j0 + b, b)
            return carry

        lax.fori_loop(1, n_groups, body, 0)

        for b in range(NBUF):
            wait_out(b)

    return k


def kernel(x, weight):
    n, s = x.shape
    B = n * s
    V, D = weight.shape
    n_tail = V % 128
    wt = weight.T
    tail = jnp.pad(weight[V - n_tail:, :], ((0, 0), (0, 128 - D)))
    t2 = _make_convert(V, D)(wt, tail)
    table = t2.reshape(V, D)
    xf = x.reshape(B)
    out = _make_gather(B, 320, 5)(table, xf)
    return out.reshape(n, s, 128)[:, :, :D]


# speedup vs baseline: 1.1965x; 1.1965x over previous
import functools

import jax
import jax.numpy as jnp
from jax import lax
from jax.experimental import pallas as pl
from jax.experimental.pallas import tpu as pltpu
from jax.experimental.pallas import tpu_sc as plsc


@functools.cache
def _make_convert(V: int, D: int):
    info = plsc.get_sparse_core_info()
    NC, NS = info.num_cores, info.num_subcores
    NW = NC * NS
    n_full = V // 128
    n_tail = V - n_full * 128
    nblk = n_full // NW
    n_rem = n_full - nblk * NW
    mesh = plsc.VectorSubcoreMesh(core_axis_name="c", subcore_axis_name="s")

    @functools.partial(
        pl.kernel,
        mesh=mesh,
        out_type=jax.ShapeDtypeStruct((V // 2, 2 * D), jnp.float32),
        scratch_types=(
            [pltpu.VMEM((D, 128), jnp.float32) for _ in range(2)]
            + [pltpu.VMEM((D, 128), jnp.float32) for _ in range(2)]
            + [pltpu.SemaphoreType.DMA] * 4
        ),
        compiler_params=pltpu.CompilerParams(
            use_tc_tiling_on_sc=True, needs_layout_passes=False,
            disable_bounds_checks=True),
    )
    def k(wt_hbm, tail_hbm, t2_hbm, a0, a1, b0, b1, si0, si1, so0, so1):
        a_v, b_v, sem_i, sem_o = (a0, a1), (b0, b1), (si0, si1), (so0, so1)
        wid = lax.axis_index("s") * NC + lax.axis_index("c")
        t0 = wid * nblk
        iotas = [16 * q + lax.iota(jnp.int32, 16) for q in range(D // 16)]

        def start_in(t, s):
            pltpu.async_copy(
                wt_hbm.at[:, pl.ds(t * 128, 128)], a_v[s], sem_i[s])

        def wait_in(s):
            pltpu.make_async_copy(
                wt_hbm.at[:, pl.ds(0, 128)], a_v[s], sem_i[s]).wait()

        def start_out(t, s):
            pltpu.async_copy(b_v[s], t2_hbm.at[pl.ds(t * 64, 64)], sem_o[s])

        def wait_out(s):
            pltpu.make_async_copy(
                b_v[s], t2_hbm.at[pl.ds(0, 64)], sem_o[s]).wait()

        def transpose_block(s):
            def row_body(r, carry):
                col0 = jnp.full((16,), 2 * r, jnp.int32)
                col1 = col0 + 1
                for q in range(128 // 16):
                    vals = plsc.load_gather(
                        a_v[s],
                        [iotas[q % (D // 16)],
                         col0 if (q * 16) < D else col1])
                    b_v[s][r, pl.ds(q * 16, 16)] = vals
                return carry
            lax.fori_loop(0, D, row_body, 0, unroll=4)

        start_in(t0, 0)

        def pair_body(p, carry):
            e = t0 + 2 * p
            start_in(e + 1, 1)
            wait_in(0)
            @pl.when(p >= 1)
            def _():
                wait_out(0)
            transpose_block(0)
            start_out(e, 0)
            @pl.when(p + 1 < nblk // 2)
            def _():
                start_in(e + 2, 0)
            wait_in(1)
            @pl.when(p >= 1)
            def _():
                wait_out(1)
            transpose_block(1)
            start_out(e + 1, 1)
            return carry

        lax.fori_loop(0, nblk // 2, pair_body, 0)
        wait_out(0)
        wait_out(1)

        if n_rem:
            @pl.when(wid < n_rem)
            def _rem():
                t = NW * nblk + wid
                start_in(t, 0)
                wait_in(0)
                transpose_block(0)
                start_out(t, 0)
                wait_out(0)

        if n_tail:
            @pl.when(wid == NW - 1)
            def _tail():
                pltpu.async_copy(
                    tail_hbm.at[:], a0.at[pl.ds(0, n_tail)], si0).wait()
                def row_body(r, carry):
                    row0 = jnp.full((16,), 2 * r, jnp.int32)
                    row1 = row0 + 1
                    for q in range(128 // 16):
                        vals = plsc.load_gather(
                            a0,
                            [row0 if (q * 16) < D else row1,
                             iotas[q % (D // 16)]])
                        b0[r, pl.ds(q * 16, 16)] = vals
                    return carry
                lax.fori_loop(0, n_tail // 2, row_body, 0)
                pltpu.async_copy(
                    b0.at[pl.ds(0, n_tail // 2)],
                    t2_hbm.at[pl.ds(n_full * 64, n_tail // 2)], si0).wait()

    return k


@functools.cache
def _make_gather(B: int, C: int, NBUF: int):
    info = plsc.get_sparse_core_info()
    NC, NS = info.num_cores, info.num_subcores
    NW = NC * NS
    assert B % (NW * C) == 0
    b_per_w = B // NW
    n_chunks = b_per_w // C
    assert n_chunks % NBUF == 0
    n_groups = n_chunks // NBUF
    mesh = plsc.VectorSubcoreMesh(core_axis_name="c", subcore_axis_name="s")

    @functools.partial(
        pl.kernel,
        mesh=mesh,
        out_type=jax.ShapeDtypeStruct((B, 128), jnp.float32),
        scratch_types=(
            [pltpu.VMEM((C,), jnp.int32) for _ in range(NBUF)]
            + [pltpu.VMEM((C, 64), jnp.float32) for _ in range(NBUF)]
            + [pltpu.SemaphoreType.DMA] * (3 * NBUF)
        ),
        compiler_params=pltpu.CompilerParams(use_tc_tiling_on_sc=False),
    )
    def k(table_hbm, idx_hbm, out_hbm, *scratch):
        idx_v = scratch[:NBUF]
        rows_v = scratch[NBUF:2 * NBUF]
        sems = scratch[2 * NBUF:]
        sem_i, sem_g, sem_o = (
            sems[:NBUF], sems[NBUF:2 * NBUF], sems[2 * NBUF:])
        wid = lax.axis_index("s") * NC + lax.axis_index("c")
        base = wid * b_per_w

        def start_idx(j, b):
            pltpu.async_copy(
                idx_hbm.at[pl.ds(base + j * C, C)], idx_v[b], sem_i[b])

        def start_gather(b):
            pltpu.make_async_copy(
                idx_hbm.at[pl.ds(0, C)], idx_v[b], sem_i[b]).wait()
            pltpu.async_copy(
                table_hbm.at[idx_v[b]], rows_v[b], sem_g[b])

        def start_out(j, b):
            pltpu.make_async_copy(
                table_hbm.at[idx_v[b]], rows_v[b], sem_g[b]).wait()
            pltpu.async_copy(
                rows_v[b],
                out_hbm.at[pl.ds(base + j * C, C), pl.ds(0, 64)], sem_o[b])

        def wait_out(b):
            pltpu.make_async_copy(
                rows_v[b],
                out_hbm.at[pl.ds(0, C), pl.ds(0, 64)], sem_o[b]).wait()

        for b in range(NBUF):
            start_idx(b, b)
        for b in range(NBUF):
            start_gather(b)
        for b in range(NBUF):
            start_out(b, b)

        def body(g, carry):
            j0 = g * NBUF
            for b in range(NBUF):
                start_idx(j0 + b, b)
            for b in range(NBUF):
                wait_out(b)
                start_gather(b)
            for b in range(NBUF):
                start_out(j0 + b, b)
            return carry

        lax.fori_loop(1, n_groups, body, 0)

        for b in range(NBUF):
            wait_out(b)

    return k


def kernel(x, weight):
    n, s = x.shape
    B = n * s
    V, D = weight.shape
    n_tail = V % 128
    wt = weight.T
    tail = jnp.pad(weight[V - n_tail:, :], ((0, 0), (0, 128 - D)))
    t2 = _make_convert(V, D)(wt, tail)
    table = t2.reshape(V, D)
    xf = x.reshape(B)
    out = _make_gather(B, 320, 5)(table, xf)
    return out.reshape(n, s, 128)[:, :, :D]


# speedup vs baseline: 1.9323x; 1.6149x over previous
import functools

import jax
import jax.numpy as jnp
from jax import lax
from jax.experimental import pallas as pl
from jax.experimental.pallas import tpu as pltpu
from jax.experimental.pallas import tpu_sc as plsc


@functools.cache
def _make_convert(V: int, D: int):
    info = plsc.get_sparse_core_info()
    NC, NS = info.num_cores, info.num_subcores
    NW = NC * NS
    n_full = V // 128
    n_tail = V - n_full * 128
    nblk = n_full // NW
    n_rem = n_full - nblk * NW
    mesh = plsc.VectorSubcoreMesh(core_axis_name="c", subcore_axis_name="s")

    @functools.partial(
        pl.kernel,
        mesh=mesh,
        out_type=jax.ShapeDtypeStruct((V // 2, 2 * D), jnp.float32),
        scratch_types=(
            [pltpu.VMEM((D, 128), jnp.float32) for _ in range(2)]
            + [pltpu.VMEM((D, 128), jnp.float32) for _ in range(2)]
            + [pltpu.SemaphoreType.DMA] * 4
        ),
        compiler_params=pltpu.CompilerParams(
            use_tc_tiling_on_sc=True, needs_layout_passes=False,
            disable_bounds_checks=True),
    )
    def k(wt_hbm, tail_hbm, t2_hbm, a0, a1, b0, b1, si0, si1, so0, so1):
        a_v, b_v, sem_i, sem_o = (a0, a1), (b0, b1), (si0, si1), (so0, so1)
        wid = lax.axis_index("s") * NC + lax.axis_index("c")
        t0 = wid * nblk
        iotas = [16 * q + lax.iota(jnp.int32, 16) for q in range(D // 16)]

        def start_in(t, s):
            pltpu.async_copy(
                wt_hbm.at[:, pl.ds(t * 128, 128)], a_v[s], sem_i[s])

        def wait_in(s):
            pltpu.make_async_copy(
                wt_hbm.at[:, pl.ds(0, 128)], a_v[s], sem_i[s]).wait()

        def start_out(t, s):
            pltpu.async_copy(b_v[s], t2_hbm.at[pl.ds(t * 64, 64)], sem_o[s])

        def wait_out(s):
            pltpu.make_async_copy(
                b_v[s], t2_hbm.at[pl.ds(0, 64)], sem_o[s]).wait()

        def transpose_block(s):
            @plsc.parallel_loop(0, D, unroll=4)
            def row_body(r):
                col0 = jnp.full((16,), 2 * r, jnp.int32)
                col1 = col0 + 1
                for q in range(128 // 16):
                    vals = plsc.load_gather(
                        a_v[s],
                        [iotas[q % (D // 16)],
                         col0 if (q * 16) < D else col1])
                    b_v[s][r, pl.ds(q * 16, 16)] = vals

        start_in(t0, 0)

        def pair_body(p, carry):
            e = t0 + 2 * p
            start_in(e + 1, 1)
            wait_in(0)
            @pl.when(p >= 1)
            def _():
                wait_out(0)
            transpose_block(0)
            start_out(e, 0)
            @pl.when(p + 1 < nblk // 2)
            def _():
                start_in(e + 2, 0)
            wait_in(1)
            @pl.when(p >= 1)
            def _():
                wait_out(1)
            transpose_block(1)
            start_out(e + 1, 1)
            return carry

        lax.fori_loop(0, nblk // 2, pair_body, 0)
        wait_out(0)
        wait_out(1)

        if n_rem:
            @pl.when(wid < n_rem)
            def _rem():
                t = NW * nblk + wid
                start_in(t, 0)
                wait_in(0)
                transpose_block(0)
                start_out(t, 0)
                wait_out(0)

        if n_tail:
            @pl.when(wid == NW - 1)
            def _tail():
                pltpu.async_copy(
                    tail_hbm.at[:], a0.at[pl.ds(0, n_tail)], si0).wait()
                def row_body(r, carry):
                    row0 = jnp.full((16,), 2 * r, jnp.int32)
                    row1 = row0 + 1
                    for q in range(128 // 16):
                        vals = plsc.load_gather(
                            a0,
                            [row0 if (q * 16) < D else row1,
                             iotas[q % (D // 16)]])
                        b0[r, pl.ds(q * 16, 16)] = vals
                    return carry
                lax.fori_loop(0, n_tail // 2, row_body, 0)
                pltpu.async_copy(
                    b0.at[pl.ds(0, n_tail // 2)],
                    t2_hbm.at[pl.ds(n_full * 64, n_tail // 2)], si0).wait()

    return k


@functools.cache
def _make_gather(B: int, C: int, NBUF: int):
    info = plsc.get_sparse_core_info()
    NC, NS = info.num_cores, info.num_subcores
    NW = NC * NS
    assert B % (NW * C) == 0
    b_per_w = B // NW
    n_chunks = b_per_w // C
    assert n_chunks % NBUF == 0
    n_groups = n_chunks // NBUF
    mesh = plsc.VectorSubcoreMesh(core_axis_name="c", subcore_axis_name="s")

    @functools.partial(
        pl.kernel,
        mesh=mesh,
        out_type=jax.ShapeDtypeStruct((B, 128), jnp.float32),
        scratch_types=(
            [pltpu.VMEM((C,), jnp.int32) for _ in range(NBUF)]
            + [pltpu.VMEM((C, 64), jnp.float32) for _ in range(NBUF)]
            + [pltpu.SemaphoreType.DMA] * (3 * NBUF)
        ),
        compiler_params=pltpu.CompilerParams(use_tc_tiling_on_sc=False),
    )
    def k(table_hbm, idx_hbm, out_hbm, *scratch):
        idx_v = scratch[:NBUF]
        rows_v = scratch[NBUF:2 * NBUF]
        sems = scratch[2 * NBUF:]
        sem_i, sem_g, sem_o = (
            sems[:NBUF], sems[NBUF:2 * NBUF], sems[2 * NBUF:])
        wid = lax.axis_index("s") * NC + lax.axis_index("c")
        base = wid * b_per_w

        def start_idx(j, b):
            pltpu.async_copy(
                idx_hbm.at[pl.ds(base + j * C, C)], idx_v[b], sem_i[b])

        def start_gather(b):
            pltpu.make_async_copy(
                idx_hbm.at[pl.ds(0, C)], idx_v[b], sem_i[b]).wait()
            pltpu.async_copy(
                table_hbm.at[idx_v[b]], rows_v[b], sem_g[b])

        def start_out(j, b):
            pltpu.make_async_copy(
                table_hbm.at[idx_v[b]], rows_v[b], sem_g[b]).wait()
            pltpu.async_copy(
                rows_v[b],
                out_hbm.at[pl.ds(base + j * C, C), pl.ds(0, 64)], sem_o[b])

        def wait_out(b):
            pltpu.make_async_copy(
                rows_v[b],
                out_hbm.at[pl.ds(0, C), pl.ds(0, 64)], sem_o[b]).wait()

        for b in range(NBUF):
            start_idx(b, b)
        for b in range(NBUF):
            start_gather(b)
        for b in range(NBUF):
            start_out(b, b)

        def body(g, carry):
            j0 = g * NBUF
            for b in range(NBUF):
                start_idx(j0 + b, b)
            for b in range(NBUF):
                wait_out(b)
                start_gather(b)
            for b in range(NBUF):
                start_out(j0 + b, b)
            return carry

        lax.fori_loop(1, n_groups, body, 0)

        for b in range(NBUF):
            wait_out(b)

    return k


def kernel(x, weight):
    n, s = x.shape
    B = n * s
    V, D = weight.shape
    n_tail = V % 128
    wt = weight.T
    tail = jnp.pad(weight[V - n_tail:, :], ((0, 0), (0, 128 - D)))
    t2 = _make_convert(V, D)(wt, tail)
    table = t2.reshape(V, D)
    xf = x.reshape(B)
    out = _make_gather(B, 320, 5)(table, xf)
    return out.reshape(n, s, 128)[:, :, :D]


# speedup vs baseline: 1.9386x; 1.0032x over previous
import functools

import jax
import jax.numpy as jnp
from jax import lax
from jax.experimental import pallas as pl
from jax.experimental.pallas import tpu as pltpu
from jax.experimental.pallas import tpu_sc as plsc


@functools.cache
def _make_convert(V: int, D: int):
    info = plsc.get_sparse_core_info()
    NC, NS = info.num_cores, info.num_subcores
    NW = NC * NS
    n_full = V // 128
    n_tail = V - n_full * 128
    nblk = n_full // NW
    n_rem = n_full - nblk * NW
    mesh = plsc.VectorSubcoreMesh(core_axis_name="c", subcore_axis_name="s")

    @functools.partial(
        pl.kernel,
        mesh=mesh,
        out_type=jax.ShapeDtypeStruct((V // 2, 2 * D), jnp.float32),
        scratch_types=(
            [pltpu.VMEM((D, 128), jnp.float32) for _ in range(2)]
            + [pltpu.VMEM((D, 128), jnp.float32) for _ in range(2)]
            + [pltpu.SemaphoreType.DMA] * 4
        ),
        compiler_params=pltpu.CompilerParams(
            use_tc_tiling_on_sc=True, needs_layout_passes=False,
            disable_bounds_checks=True),
    )
    def k(wt_hbm, tail_hbm, t2_hbm, a0, a1, b0, b1, si0, si1, so0, so1):
        a_v, b_v, sem_i, sem_o = (a0, a1), (b0, b1), (si0, si1), (so0, so1)
        wid = lax.axis_index("s") * NC + lax.axis_index("c")
        t0 = wid * nblk
        iotas = [16 * q + lax.iota(jnp.int32, 16) for q in range(D // 16)]

        def start_in(t, s):
            pltpu.async_copy(
                wt_hbm.at[:, pl.ds(t * 128, 128)], a_v[s], sem_i[s])

        def wait_in(s):
            pltpu.make_async_copy(
                wt_hbm.at[:, pl.ds(0, 128)], a_v[s], sem_i[s]).wait()

        def start_out(t, s):
            pltpu.async_copy(b_v[s], t2_hbm.at[pl.ds(t * 64, 64)], sem_o[s])

        def wait_out(s):
            pltpu.make_async_copy(
                b_v[s], t2_hbm.at[pl.ds(0, 64)], sem_o[s]).wait()

        def transpose_block(s):
            @plsc.parallel_loop(0, D, unroll=8)
            def row_body(r):
                col0 = jnp.full((16,), 2 * r, jnp.int32)
                col1 = col0 + 1
                for q in range(128 // 16):
                    vals = plsc.load_gather(
                        a_v[s],
                        [iotas[q % (D // 16)],
                         col0 if (q * 16) < D else col1])
                    b_v[s][r, pl.ds(q * 16, 16)] = vals

        start_in(t0, 0)

        def pair_body(p, carry):
            e = t0 + 2 * p
            start_in(e + 1, 1)
            wait_in(0)
            @pl.when(p >= 1)
            def _():
                wait_out(0)
            transpose_block(0)
            start_out(e, 0)
            @pl.when(p + 1 < nblk // 2)
            def _():
                start_in(e + 2, 0)
            wait_in(1)
            @pl.when(p >= 1)
            def _():
                wait_out(1)
            transpose_block(1)
            start_out(e + 1, 1)
            return carry

        lax.fori_loop(0, nblk // 2, pair_body, 0)
        wait_out(0)
        wait_out(1)

        if n_rem:
            @pl.when(wid < n_rem)
            def _rem():
                t = NW * nblk + wid
                start_in(t, 0)
                wait_in(0)
                transpose_block(0)
                start_out(t, 0)
                wait_out(0)

        if n_tail:
            @pl.when(wid == NW - 1)
            def _tail():
                pltpu.async_copy(
                    tail_hbm.at[:], a0.at[pl.ds(0, n_tail)], si0).wait()
                def row_body(r, carry):
                    row0 = jnp.full((16,), 2 * r, jnp.int32)
                    row1 = row0 + 1
                    for q in range(128 // 16):
                        vals = plsc.load_gather(
                            a0,
                            [row0 if (q * 16) < D else row1,
                             iotas[q % (D // 16)]])
                        b0[r, pl.ds(q * 16, 16)] = vals
                    return carry
                lax.fori_loop(0, n_tail // 2, row_body, 0)
                pltpu.async_copy(
                    b0.at[pl.ds(0, n_tail // 2)],
                    t2_hbm.at[pl.ds(n_full * 64, n_tail // 2)], si0).wait()

    return k


@functools.cache
def _make_gather(B: int, C: int, NBUF: int):
    info = plsc.get_sparse_core_info()
    NC, NS = info.num_cores, info.num_subcores
    NW = NC * NS
    assert B % (NW * C) == 0
    b_per_w = B // NW
    n_chunks = b_per_w // C
    assert n_chunks % NBUF == 0
    n_groups = n_chunks // NBUF
    mesh = plsc.VectorSubcoreMesh(core_axis_name="c", subcore_axis_name="s")

    @functools.partial(
        pl.kernel,
        mesh=mesh,
        out_type=jax.ShapeDtypeStruct((B, 128), jnp.float32),
        scratch_types=(
            [pltpu.VMEM((C,), jnp.int32) for _ in range(NBUF)]
            + [pltpu.VMEM((C, 64), jnp.float32) for _ in range(NBUF)]
            + [pltpu.SemaphoreType.DMA] * (3 * NBUF)
        ),
        compiler_params=pltpu.CompilerParams(use_tc_tiling_on_sc=False),
    )
    def k(table_hbm, idx_hbm, out_hbm, *scratch):
        idx_v = scratch[:NBUF]
        rows_v = scratch[NBUF:2 * NBUF]
        sems = scratch[2 * NBUF:]
        sem_i, sem_g, sem_o = (
            sems[:NBUF], sems[NBUF:2 * NBUF], sems[2 * NBUF:])
        wid = lax.axis_index("s") * NC + lax.axis_index("c")
        base = wid * b_per_w

        def start_idx(j, b):
            pltpu.async_copy(
                idx_hbm.at[pl.ds(base + j * C, C)], idx_v[b], sem_i[b])

        def start_gather(b):
            pltpu.make_async_copy(
                idx_hbm.at[pl.ds(0, C)], idx_v[b], sem_i[b]).wait()
            pltpu.async_copy(
                table_hbm.at[idx_v[b]], rows_v[b], sem_g[b])

        def start_out(j, b):
            pltpu.make_async_copy(
                table_hbm.at[idx_v[b]], rows_v[b], sem_g[b]).wait()
            pltpu.async_copy(
                rows_v[b],
                out_hbm.at[pl.ds(base + j * C, C), pl.ds(0, 64)], sem_o[b])

        def wait_out(b):
            pltpu.make_async_copy(
                rows_v[b],
                out_hbm.at[pl.ds(0, C), pl.ds(0, 64)], sem_o[b]).wait()

        for b in range(NBUF):
            start_idx(b, b)
        for b in range(NBUF):
            start_gather(b)
        for b in range(NBUF):
            start_out(b, b)

        def body(g, carry):
            j0 = g * NBUF
            for b in range(NBUF):
                start_idx(j0 + b, b)
            for b in range(NBUF):
                wait_out(b)
                start_gather(b)
            for b in range(NBUF):
                start_out(j0 + b, b)
            return carry

        lax.fori_loop(1, n_groups, body, 0)

        for b in range(NBUF):
            wait_out(b)

    return k


def kernel(x, weight):
    n, s = x.shape
    B = n * s
    V, D = weight.shape
    n_tail = V % 128
    wt = weight.T
    tail = jnp.pad(weight[V - n_tail:, :], ((0, 0), (0, 128 - D)))
    t2 = _make_convert(V, D)(wt, tail)
    table = t2.reshape(V, D)
    xf = x.reshape(B)
    out = _make_gather(B, 320, 5)(table, xf)
    return out.reshape(n, s, 128)[:, :, :D]


# speedup vs baseline: 2.5610x; 1.3210x over previous
import functools

import jax
import jax.numpy as jnp
from jax import lax
from jax.experimental import pallas as pl
from jax.experimental.pallas import tpu as pltpu
from jax.experimental.pallas import tpu_sc as plsc


@functools.cache
def _make_convert(V: int, D: int):
    info = plsc.get_sparse_core_info()
    NC, NS = info.num_cores, info.num_subcores
    NW = NC * NS
    n_full = V // 128
    n_tail = V - n_full * 128
    nblk = n_full // NW
    n_rem = n_full - nblk * NW
    mesh = plsc.VectorSubcoreMesh(core_axis_name="c", subcore_axis_name="s")

    @functools.partial(
        pl.kernel,
        mesh=mesh,
        out_type=jax.ShapeDtypeStruct((V // 2, 2 * D), jnp.float32),
        scratch_types=(
            [pltpu.VMEM((D, 128), jnp.float32) for _ in range(2)]
            + [pltpu.VMEM((D, 128), jnp.float32) for _ in range(2)]
            + [pltpu.SemaphoreType.DMA] * 4
        ),
        compiler_params=pltpu.CompilerParams(
            use_tc_tiling_on_sc=True, needs_layout_passes=False,
            disable_bounds_checks=True),
    )
    def k(wt_hbm, tail_hbm, t2_hbm, a0, a1, b0, b1, si0, si1, so0, so1):
        a_v, b_v, sem_i, sem_o = (a0, a1), (b0, b1), (si0, si1), (so0, so1)
        wid = lax.axis_index("s") * NC + lax.axis_index("c")
        t0 = wid * nblk
        i16 = lax.iota(jnp.int32, 16)
        iotas = [16 * q + i16 for q in range(D // 16)]
        pk = [(i16 + k) % 16 for k in range(16)]
        rk = [p // 2 for p in pk]
        ck = [(p % 2) * 64 + i16 for p in pk]

        def start_in(t, s):
            pltpu.async_copy(
                wt_hbm.at[:, pl.ds(t * 128, 128)], a_v[s], sem_i[s])

        def wait_in(s):
            pltpu.make_async_copy(
                wt_hbm.at[:, pl.ds(0, 128)], a_v[s], sem_i[s]).wait()

        def start_out(t, s):
            pltpu.async_copy(b_v[s], t2_hbm.at[pl.ds(t * 64, 64)], sem_o[s])

        def wait_out(s):
            pltpu.make_async_copy(
                b_v[s], t2_hbm.at[pl.ds(0, 64)], sem_o[s]).wait()

        def transpose_block(s):
            for a in range(D // 16):
                row_a = iotas[a]
                ca = [c_ + 16 * a for c_ in ck]

                @plsc.parallel_loop(0, 8, unroll=2)
                def col_body(c):
                    for k in range(16):
                        vals = plsc.load_gather(
                            a_v[s], [row_a, pk[k] + 16 * c])
                        plsc.store_scatter(
                            b_v[s], [rk[k] + 8 * c, ca[k]], vals)

        start_in(t0, 0)

        def pair_body(p, carry):
            e = t0 + 2 * p
            start_in(e + 1, 1)
            wait_in(0)
            @pl.when(p >= 1)
            def _():
                wait_out(0)
            transpose_block(0)
            start_out(e, 0)
            @pl.when(p + 1 < nblk // 2)
            def _():
                start_in(e + 2, 0)
            wait_in(1)
            @pl.when(p >= 1)
            def _():
                wait_out(1)
            transpose_block(1)
            start_out(e + 1, 1)
            return carry

        lax.fori_loop(0, nblk // 2, pair_body, 0)
        wait_out(0)
        wait_out(1)

        if n_rem:
            @pl.when(wid < n_rem)
            def _rem():
                t = NW * nblk + wid
                start_in(t, 0)
                wait_in(0)
                transpose_block(0)
                start_out(t, 0)
                wait_out(0)

        if n_tail:
            @pl.when(wid == NW - 1)
            def _tail():
                pltpu.async_copy(
                    tail_hbm.at[:], a0.at[pl.ds(0, n_tail)], si0).wait()
                def row_body(r, carry):
                    row0 = jnp.full((16,), 2 * r, jnp.int32)
                    row1 = row0 + 1
                    for q in range(128 // 16):
                        vals = plsc.load_gather(
                            a0,
                            [row0 if (q * 16) < D else row1,
                             iotas[q % (D // 16)]])
                        b0[r, pl.ds(q * 16, 16)] = vals
                    return carry
                lax.fori_loop(0, n_tail // 2, row_body, 0)
                pltpu.async_copy(
                    b0.at[pl.ds(0, n_tail // 2)],
                    t2_hbm.at[pl.ds(n_full * 64, n_tail // 2)], si0).wait()

    return k


@functools.cache
def _make_gather(B: int, C: int, NBUF: int):
    info = plsc.get_sparse_core_info()
    NC, NS = info.num_cores, info.num_subcores
    NW = NC * NS
    assert B % (NW * C) == 0
    b_per_w = B // NW
    n_chunks = b_per_w // C
    assert n_chunks % NBUF == 0
    n_groups = n_chunks // NBUF
    mesh = plsc.VectorSubcoreMesh(core_axis_name="c", subcore_axis_name="s")

    @functools.partial(
        pl.kernel,
        mesh=mesh,
        out_type=jax.ShapeDtypeStruct((B, 128), jnp.float32),
        scratch_types=(
            [pltpu.VMEM((C,), jnp.int32) for _ in range(NBUF)]
            + [pltpu.VMEM((C, 64), jnp.float32) for _ in range(NBUF)]
            + [pltpu.SemaphoreType.DMA] * (3 * NBUF)
        ),
        compiler_params=pltpu.CompilerParams(use_tc_tiling_on_sc=False),
    )
    def k(table_hbm, idx_hbm, out_hbm, *scratch):
        idx_v = scratch[:NBUF]
        rows_v = scratch[NBUF:2 * NBUF]
        sems = scratch[2 * NBUF:]
        sem_i, sem_g, sem_o = (
            sems[:NBUF], sems[NBUF:2 * NBUF], sems[2 * NBUF:])
        wid = lax.axis_index("s") * NC + lax.axis_index("c")
        base = wid * b_per_w

        def start_idx(j, b):
            pltpu.async_copy(
                idx_hbm.at[pl.ds(base + j * C, C)], idx_v[b], sem_i[b])

        def start_gather(b):
            pltpu.make_async_copy(
                idx_hbm.at[pl.ds(0, C)], idx_v[b], sem_i[b]).wait()
            pltpu.async_copy(
                table_hbm.at[idx_v[b]], rows_v[b], sem_g[b])

        def start_out(j, b):
            pltpu.make_async_copy(
                table_hbm.at[idx_v[b]], rows_v[b], sem_g[b]).wait()
            pltpu.async_copy(
                rows_v[b],
                out_hbm.at[pl.ds(base + j * C, C), pl.ds(0, 64)], sem_o[b])

        def wait_out(b):
            pltpu.make_async_copy(
                rows_v[b],
                out_hbm.at[pl.ds(0, C), pl.ds(0, 64)], sem_o[b]).wait()

        for b in range(NBUF):
            start_idx(b, b)
        for b in range(NBUF):
            start_gather(b)
        for b in range(NBUF):
            start_out(b, b)

        def body(g, carry):
            j0 = g * NBUF
            for b in range(NBUF):
                start_idx(j0 + b, b)
            for b in range(NBUF):
                wait_out(b)
                start_gather(b)
            for b in range(NBUF):
                start_out(j0 + b, b)
            return carry

        lax.fori_loop(1, n_groups, body, 0)

        for b in range(NBUF):
            wait_out(b)

    return k


def kernel(x, weight):
    n, s = x.shape
    B = n * s
    V, D = weight.shape
    n_tail = V % 128
    wt = weight.T
    tail = jnp.pad(weight[V - n_tail:, :], ((0, 0), (0, 128 - D)))
    t2 = _make_convert(V, D)(wt, tail)
    table = t2.reshape(V, D)
    xf = x.reshape(B)
    out = _make_gather(B, 320, 5)(table, xf)
    return out.reshape(n, s, 128)[:, :, :D]


# speedup vs baseline: 3.8075x; 1.4867x over previous
import functools

import jax
import jax.numpy as jnp
from jax import lax
from jax.experimental import pallas as pl
from jax.experimental.pallas import tpu as pltpu
from jax.experimental.pallas import tpu_sc as plsc


@functools.cache
def _make_convert(V: int, D: int):
    info = plsc.get_sparse_core_info()
    NC, NS = info.num_cores, info.num_subcores
    NW = NC * NS
    n_full = V // 128
    n_tail = V - n_full * 128
    nblk = n_full // NW
    n_rem = n_full - nblk * NW
    mesh = plsc.VectorSubcoreMesh(core_axis_name="c", subcore_axis_name="s")

    @functools.partial(
        pl.kernel,
        mesh=mesh,
        out_type=jax.ShapeDtypeStruct((V // 2, 2 * D), jnp.float32),
        scratch_types=(
            [pltpu.VMEM((D, 128), jnp.float32) for _ in range(2)]
            + [pltpu.VMEM((D, 128), jnp.float32) for _ in range(2)]
            + [pltpu.SemaphoreType.DMA] * 4
        ),
        compiler_params=pltpu.CompilerParams(
            use_tc_tiling_on_sc=True, needs_layout_passes=False,
            disable_bounds_checks=True),
    )
    def k(wt_hbm, tail_hbm, t2_hbm, a0, a1, b0, b1, si0, si1, so0, so1):
        a_v, b_v, sem_i, sem_o = (a0, a1), (b0, b1), (si0, si1), (so0, so1)
        wid = lax.axis_index("s") * NC + lax.axis_index("c")
        t0 = wid * nblk
        i16 = lax.iota(jnp.int32, 16)
        iotas = [16 * q + i16 for q in range(D // 16)]
        pk = [(i16 + k) % 16 for k in range(16)]
        rk = [p // 2 for p in pk]
        ck = [(p % 2) * 64 + i16 for p in pk]

        def start_in(t, s):
            pltpu.async_copy(
                wt_hbm.at[:, pl.ds(t * 128, 128)], a_v[s], sem_i[s])

        def wait_in(s):
            pltpu.make_async_copy(
                wt_hbm.at[:, pl.ds(0, 128)], a_v[s], sem_i[s]).wait()

        def start_out(t, s):
            pltpu.async_copy(b_v[s], t2_hbm.at[pl.ds(t * 64, 64)], sem_o[s])

        def wait_out(s):
            pltpu.make_async_copy(
                b_v[s], t2_hbm.at[pl.ds(0, 64)], sem_o[s]).wait()

        def transpose_block(s):
            for a in range(D // 16):
                row_a = iotas[a]
                ca = [c_ + 16 * a for c_ in ck]

                @plsc.parallel_loop(0, 8, unroll=4)
                def col_body(c):
                    for k in range(16):
                        vals = plsc.load_gather(
                            a_v[s], [row_a, pk[k] + 16 * c])
                        plsc.store_scatter(
                            b_v[s], [rk[k] + 8 * c, ca[k]], vals)

        start_in(t0, 0)

        def pair_body(p, carry):
            e = t0 + 2 * p
            start_in(e + 1, 1)
            wait_in(0)
            @pl.when(p >= 1)
            def _():
                wait_out(0)
            transpose_block(0)
            start_out(e, 0)
            @pl.when(p + 1 < nblk // 2)
            def _():
                start_in(e + 2, 0)
            wait_in(1)
            @pl.when(p >= 1)
            def _():
                wait_out(1)
            transpose_block(1)
            start_out(e + 1, 1)
            return carry

        lax.fori_loop(0, nblk // 2, pair_body, 0)
        wait_out(0)
        wait_out(1)

        if n_rem:
            @pl.when(wid < n_rem)
            def _rem():
                t = NW * nblk + wid
                start_in(t, 0)
                wait_in(0)
                transpose_block(0)
                start_out(t, 0)
                wait_out(0)

        if n_tail:
            @pl.when(wid == NW - 1)
            def _tail():
                pltpu.async_copy(
                    tail_hbm.at[:], a0.at[pl.ds(0, n_tail)], si0).wait()
                def row_body(r, carry):
                    row0 = jnp.full((16,), 2 * r, jnp.int32)
                    row1 = row0 + 1
                    for q in range(128 // 16):
                        vals = plsc.load_gather(
                            a0,
                            [row0 if (q * 16) < D else row1,
                             iotas[q % (D // 16)]])
                        b0[r, pl.ds(q * 16, 16)] = vals
                    return carry
                lax.fori_loop(0, n_tail // 2, row_body, 0)
                pltpu.async_copy(
                    b0.at[pl.ds(0, n_tail // 2)],
                    t2_hbm.at[pl.ds(n_full * 64, n_tail // 2)], si0).wait()

    return k


@functools.cache
def _make_gather(B: int, C: int, NBUF: int):
    info = plsc.get_sparse_core_info()
    NC, NS = info.num_cores, info.num_subcores
    NW = NC * NS
    assert B % (NW * C) == 0
    b_per_w = B // NW
    n_chunks = b_per_w // C
    assert n_chunks % NBUF == 0
    n_groups = n_chunks // NBUF
    mesh = plsc.VectorSubcoreMesh(core_axis_name="c", subcore_axis_name="s")

    @functools.partial(
        pl.kernel,
        mesh=mesh,
        out_type=jax.ShapeDtypeStruct((B, 128), jnp.float32),
        scratch_types=(
            [pltpu.VMEM((C,), jnp.int32) for _ in range(NBUF)]
            + [pltpu.VMEM((C, 64), jnp.float32) for _ in range(NBUF)]
            + [pltpu.SemaphoreType.DMA] * (3 * NBUF)
        ),
        compiler_params=pltpu.CompilerParams(use_tc_tiling_on_sc=False),
    )
    def k(table_hbm, idx_hbm, out_hbm, *scratch):
        idx_v = scratch[:NBUF]
        rows_v = scratch[NBUF:2 * NBUF]
        sems = scratch[2 * NBUF:]
        sem_i, sem_g, sem_o = (
            sems[:NBUF], sems[NBUF:2 * NBUF], sems[2 * NBUF:])
        wid = lax.axis_index("s") * NC + lax.axis_index("c")
        base = wid * b_per_w

        def start_idx(j, b):
            pltpu.async_copy(
                idx_hbm.at[pl.ds(base + j * C, C)], idx_v[b], sem_i[b])

        def start_gather(b):
            pltpu.make_async_copy(
                idx_hbm.at[pl.ds(0, C)], idx_v[b], sem_i[b]).wait()
            pltpu.async_copy(
                table_hbm.at[idx_v[b]], rows_v[b], sem_g[b])

        def start_out(j, b):
            pltpu.make_async_copy(
                table_hbm.at[idx_v[b]], rows_v[b], sem_g[b]).wait()
            pltpu.async_copy(
                rows_v[b],
                out_hbm.at[pl.ds(base + j * C, C), pl.ds(0, 64)], sem_o[b])

        def wait_out(b):
            pltpu.make_async_copy(
                rows_v[b],
                out_hbm.at[pl.ds(0, C), pl.ds(0, 64)], sem_o[b]).wait()

        for b in range(NBUF):
            start_idx(b, b)
        for b in range(NBUF):
            start_gather(b)
        for b in range(NBUF):
            start_out(b, b)

        def body(g, carry):
            j0 = g * NBUF
            for b in range(NBUF):
                start_idx(j0 + b, b)
            for b in range(NBUF):
                wait_out(b)
                start_gather(b)
            for b in range(NBUF):
                start_out(j0 + b, b)
            return carry

        lax.fori_loop(1, n_groups, body, 0)

        for b in range(NBUF):
            wait_out(b)

    return k


def kernel(x, weight):
    n, s = x.shape
    B = n * s
    V, D = weight.shape
    n_tail = V % 128
    wt = weight.T
    tail = jnp.pad(weight[V - n_tail:, :], ((0, 0), (0, 128 - D)))
    t2 = _make_convert(V, D)(wt, tail)
    table = t2.reshape(V, D)
    xf = x.reshape(B)
    out = _make_gather(B, 320, 5)(table, xf)
    return out.reshape(n, s, 128)[:, :, :D]
